# baseline (speedup 1.0000x reference)
import functools

import jax
import jax.numpy as jnp
from jax import lax
from jax.experimental import pallas as pl
from jax.experimental.pallas import tpu as pltpu
from jax.experimental.pallas import tpu_sc as plsc

N = 10000
D = 128
E = 320000
FPT = 4
CHUNK = 16000
NCHUNK = (E * FPT) // CHUNK

_CONTRACT_00 = (((0,), (0,)), ((), ()))
_CONTRACT_01 = (((0,), (1,)), ((), ()))


def _dot(a, b, dims):
    return lax.dot_general(a, b, dims, preferred_element_type=jnp.float32)


def _tc1_body(x_ref, wp_ref, bp_ref, ws_ref, bs_ref, m_ref, s_ref):
    x = x_ref[...]
    m_ref[...] = jnp.maximum(_dot(wp_ref[...], x, _CONTRACT_01) + bp_ref[...], 0.0)
    s_ref[...] = _dot(ws_ref[...], x, _CONTRACT_01) + bs_ref[...]


def _tc2_body(s1_ref, agg_ref, wn_ref, wp2_ref, bp2_ref, h_ref, m2_ref):
    h = s1_ref[...] + _dot(wn_ref[...], agg_ref[...], _CONTRACT_00)
    h_ref[...] = h
    m2_ref[...] = jnp.maximum(_dot(wp2_ref[...], h, _CONTRACT_00) + bp2_ref[...], 0.0)


def _tc3_body(h_ref, agg2_ref, ws2_ref, wn2_ref, bs2_ref, out_ref):
    out_ref[...] = (_dot(h_ref[...], ws2_ref[...], _CONTRACT_00)
                    + _dot(agg2_ref[...], wn2_ref[...], _CONTRACT_00)
                    + bs2_ref[...])


@functools.partial(
    pl.kernel,
    out_type=jax.ShapeDtypeStruct((D * N,), jnp.float32),
    mesh=plsc.VectorSubcoreMesh(core_axis_name="c", subcore_axis_name="s",
                                num_cores=2),
    compiler_params=pltpu.CompilerParams(needs_layout_passes=False),
    scratch_types=[
        pltpu.VMEM((FPT * N,), jnp.float32),
        pltpu.VMEM((FPT * N,), jnp.float32),
        pltpu.VMEM((CHUNK,), jnp.int32),
        pltpu.VMEM((CHUNK,), jnp.int32),
    ],
)
def _segmax(mT, srcr, dstr, out, m_v, agg_v, s_v, d_v):
    wid = lax.axis_index("s") * 2 + lax.axis_index("c")
    e0 = wid * (FPT * N)
    pltpu.sync_copy(mT.at[pl.ds(e0, FPT * N)], m_v)

    zeros16 = jnp.zeros((16,), jnp.float32)

    def _zero_body(i, _):
        agg_v[pl.ds(i * 16, 16)] = zeros16
        return 0
    lax.fori_loop(0, FPT * N // 16, _zero_body, 0)

    def _chunk_body(g, _):
        base = g * CHUNK
        pltpu.sync_copy(srcr.at[pl.ds(base, CHUNK)], s_v)
        pltpu.sync_copy(dstr.at[pl.ds(base, CHUNK)], d_v)

        def _group_body(i, _):
            sl = s_v[pl.ds(i * 16, 16)]
            dl = d_v[pl.ds(i * 16, 16)]
            val = plsc.load_gather(m_v, [sl])
            cur = plsc.load_gather(agg_v, [dl])
            plsc.store_scatter(agg_v, [dl], jnp.maximum(cur, val))
            chk = plsc.load_gather(agg_v, [dl])
            pend = chk < val

            def _wcond(p):
                return jnp.any(p)

            def _wbody(p):
                cur2 = plsc.load_gather(agg_v, [dl])
                plsc.store_scatter(agg_v, [dl], jnp.maximum(cur2, val),
                                   mask=p)
                chk2 = plsc.load_gather(agg_v, [dl])
                return p & (chk2 < val)

            lax.while_loop(_wcond, _wbody, pend)
            return 0

        lax.fori_loop(0, CHUNK // 16, _group_body, 0)
        return 0

    lax.fori_loop(0, NCHUNK, _chunk_body, 0)
    pltpu.sync_copy(agg_v, out.at[pl.ds(e0, FPT * N)])


def kernel(x, edge_index, Wp1, bp1, Ws1, bs1, Wn1, Wp2, bp2, Ws2, bs2, Wn2):
    src = edge_index[0].astype(jnp.int32)
    dst = edge_index[1].astype(jnp.int32)
    foff = jnp.arange(FPT, dtype=jnp.int32)[None, :] * N
    srcr = (src[:, None] + foff).reshape(-1)
    dstr = (dst[:, None] + foff).reshape(-1)

    m1T, s1T = pl.pallas_call(
        _tc1_body,
        out_shape=[jax.ShapeDtypeStruct((D, N), jnp.float32)] * 2,
    )(x, Wp1, bp1.reshape(D, 1), Ws1, bs1.reshape(D, 1))

    agg1T = _segmax(m1T.reshape(-1), srcr, dstr).reshape(D, N)

    h1T, m2T = pl.pallas_call(
        _tc2_body,
        out_shape=[jax.ShapeDtypeStruct((D, N), jnp.float32)] * 2,
    )(s1T, agg1T, Wn1, Wp2, bp2.reshape(D, 1))

    agg2T = _segmax(m2T.reshape(-1), srcr, dstr).reshape(D, N)

    out = pl.pallas_call(
        _tc3_body,
        out_shape=jax.ShapeDtypeStruct((N, D), jnp.float32),
    )(h1T, agg2T, Ws2, Wn2, bs2.reshape(1, D))
    return out
